# Initial kernel scaffold; baseline (speedup 1.0000x reference)
import jax
import jax.numpy as jnp
from jax.experimental import pallas as pl


def kernel(hidden_states, router_w, router_b, expert_gate, expert_up, expert_down, shared_conv_w, shared_up, shared_down):
    raise NotImplementedError("write your pallas kernel here")


# speedup vs baseline: 1.2179x; 1.2179x over previous
import jax
import jax.numpy as jnp
from jax.experimental import pallas as pl
from jax.experimental.pallas import tpu as pltpu

B, S, H = 1, 2048, 1024
E, TOPK = 8, 2
DFF = 512
KSZ = 4
T = B * S


def _router_body(x_ref, rw_ref, rb_ref, comb_ref):
    x = x_ref[...].astype(jnp.bfloat16)
    logits = jnp.dot(x, rw_ref[...].astype(jnp.bfloat16),
                     preferred_element_type=jnp.float32)
    logits = logits + rb_ref[...]
    m = jnp.max(logits, axis=1, keepdims=True)
    ex = jnp.exp(logits - m)
    p = ex / jnp.sum(ex, axis=1, keepdims=True)
    lane = jax.lax.broadcasted_iota(jnp.int32, (T, E), 1)
    v1 = jnp.max(p, axis=1, keepdims=True)
    i1 = jnp.min(jnp.where(p == v1, lane, E), axis=1, keepdims=True)
    m1 = lane == i1
    pm = jnp.where(m1, -1.0, p)
    v2 = jnp.max(pm, axis=1, keepdims=True)
    i2 = jnp.min(jnp.where(pm == v2, lane, E), axis=1, keepdims=True)
    m2 = lane == i2
    s = v1 + v2
    comb_ref[...] = (jnp.where(m1, v1, 0.0) + jnp.where(m2, v2, 0.0)) / s


def _expert_body(x_ref, comb_ref, wg_ref, wu_ref, wd_ref, out_ref):
    e = pl.program_id(0)
    x = x_ref[...]
    g = jnp.dot(x, wg_ref[0], preferred_element_type=jnp.float32)
    u = jnp.dot(x, wu_ref[0], preferred_element_type=jnp.float32)
    h = (g / (1.0 + jnp.exp(-g))) * u
    eo = jnp.dot(h.astype(jnp.bfloat16), wd_ref[0],
                 preferred_element_type=jnp.float32)
    comb = comb_ref[...]
    lane = jax.lax.broadcasted_iota(jnp.int32, (T, E), 1)
    col = jnp.sum(jnp.where(lane == e, comb, 0.0), axis=1, keepdims=True)
    contrib = eo * col

    @pl.when(e == 0)
    def _():
        out_ref[...] = contrib

    @pl.when(e != 0)
    def _():
        out_ref[...] += contrib


def _shared_body(xcat_ref, x_ref, wcat_ref, sup_ref, sdn_ref, routed_ref, out_ref):
    gate = jnp.dot(xcat_ref[...], wcat_ref[...], preferred_element_type=jnp.float32)
    up = jnp.dot(x_ref[...], sup_ref[...], preferred_element_type=jnp.float32)
    h = (gate / (1.0 + jnp.exp(-gate))) * up
    so = jnp.dot(h.astype(jnp.bfloat16), sdn_ref[...],
                 preferred_element_type=jnp.float32)
    out_ref[...] = so + routed_ref[...]


def kernel(hidden_states, router_w, router_b, expert_gate, expert_up,
           expert_down, shared_conv_w, shared_up, shared_down):
    x = hidden_states.reshape(T, H)
    x16 = x.astype(jnp.bfloat16)

    comb = pl.pallas_call(
        _router_body,
        out_shape=jax.ShapeDtypeStruct((T, E), jnp.float32),
    )(x, router_w, router_b.reshape(1, E))

    routed = pl.pallas_call(
        _expert_body,
        grid=(E,),
        in_specs=[
            pl.BlockSpec((T, H), lambda e: (0, 0)),
            pl.BlockSpec((T, E), lambda e: (0, 0)),
            pl.BlockSpec((1, H, DFF), lambda e: (e, 0, 0)),
            pl.BlockSpec((1, H, DFF), lambda e: (e, 0, 0)),
            pl.BlockSpec((1, DFF, H), lambda e: (e, 0, 0)),
        ],
        out_specs=pl.BlockSpec((T, H), lambda e: (0, 0)),
        out_shape=jax.ShapeDtypeStruct((T, H), jnp.float32),
    )(x16, comb,
      expert_gate.astype(jnp.bfloat16),
      expert_up.astype(jnp.bfloat16),
      expert_down.astype(jnp.bfloat16))

    xp = jnp.pad(x16, ((KSZ - 1, 0), (0, 0)))
    xcat = jnp.concatenate([xp[k:T + k] for k in range(KSZ)], axis=1)
    wcat = jnp.concatenate(
        [shared_conv_w[:, :, k].T for k in range(KSZ)], axis=0
    ).astype(jnp.bfloat16)

    TM = 512
    out = pl.pallas_call(
        _shared_body,
        grid=(T // TM,),
        in_specs=[
            pl.BlockSpec((TM, KSZ * H), lambda i: (i, 0)),
            pl.BlockSpec((TM, H), lambda i: (i, 0)),
            pl.BlockSpec((KSZ * H, DFF), lambda i: (0, 0)),
            pl.BlockSpec((H, DFF), lambda i: (0, 0)),
            pl.BlockSpec((DFF, H), lambda i: (0, 0)),
            pl.BlockSpec((TM, H), lambda i: (i, 0)),
        ],
        out_specs=pl.BlockSpec((TM, H), lambda i: (i, 0)),
        out_shape=jax.ShapeDtypeStruct((T, H), jnp.float32),
    )(xcat, x16, wcat,
      shared_up.astype(jnp.bfloat16), shared_down.astype(jnp.bfloat16),
      routed)

    return out.reshape(B, S, H)
